# baseline (speedup 1.0000x reference)
import functools

import jax
import jax.numpy as jnp
from jax import lax
from jax.experimental import pallas as pl
from jax.experimental.pallas import tpu as pltpu


def _smoe_kernel(x_hbm, rw_ref, wall_ref, sel_ref, map_ref, idx_ref,
                 xbuf, pp_ref, sems, *,
                 jblocks, nsteps, seg, slab, pad, W, E, K, C):
    s = pl.program_id(0)
    edge = slab - pad

    nchunk = sems.shape[1]
    cc = C // nchunk

    def dma(t, go):
        b = t // jblocks
        j = t - b * jblocks
        slot = t % 2
        first = j == 0
        last = (j == jblocks - 1) & jnp.logical_not(first)

        @pl.when(first)
        def _():
            for i in range(nchunk):
                cp = pltpu.make_async_copy(
                    x_hbm.at[b, pl.ds(i * cc, cc), pl.ds(0, edge)],
                    xbuf.at[slot, pl.ds(i * cc, cc), pl.ds(pad, edge)],
                    sems.at[slot, i])
                cp.start() if go else cp.wait()

        @pl.when(last)
        def _():
            for i in range(nchunk):
                cp = pltpu.make_async_copy(
                    x_hbm.at[b, pl.ds(i * cc, cc), pl.ds(pl.multiple_of(jnp.maximum(j * seg - pad, 0), 128), edge)],
                    xbuf.at[slot, pl.ds(i * cc, cc), pl.ds(0, edge)],
                    sems.at[slot, i])
                cp.start() if go else cp.wait()

        @pl.when(jnp.logical_not(first | last))
        def _():
            for i in range(nchunk):
                cp = pltpu.make_async_copy(
                    x_hbm.at[b, pl.ds(i * cc, cc), pl.ds(pl.multiple_of(jnp.maximum(j * seg - pad, 0), 128), slab)],
                    xbuf.at[slot, pl.ds(i * cc, cc)],
                    sems.at[slot, i])
                cp.start() if go else cp.wait()

    @pl.when(s == 0)
    def _prologue():
        dma(0, True)

    @pl.when(s + 1 < nsteps)
    def _prefetch():
        dma(s + 1, True)

    dma(s, False)

    xs = xbuf[s % 2].astype(jnp.bfloat16)
    pp_ref[...] = jnp.dot(wall_ref[...], xs,
                          preferred_element_type=jnp.float32
                          ).astype(jnp.bfloat16)

    j = s % jblocks
    pio = lax.broadcasted_iota(jnp.int32, (1, seg), 1)
    wpos = pio % W
    top_bad = (pio < W) & (j == 0)
    bot_bad = (pio >= seg - W) & (j == jblocks - 1)
    acc = None
    for t in range(9):
        dy, dx = divmod(t, 3)
        off = dy * W + dx + pad - W - 1
        part = pp_ref[t * E:(t + 1) * E, off:off + seg]
        if dx == 0:
            part = jnp.where(wpos == 0, 0.0, part)
        elif dx == 2:
            part = jnp.where(wpos == W - 1, 0.0, part)
        if dy == 0:
            part = jnp.where(top_bad, 0.0, part)
        elif dy == 2:
            part = jnp.where(bot_bad, 0.0, part)
        acc = part if acc is None else acc + part

    rw = rw_ref[0]
    iota = lax.broadcasted_iota(jnp.int32, (E, seg), 0)
    neg_inf = jnp.float32(-jnp.inf)
    for k in range(K):
        m = jnp.max(rw, axis=0)
        eq = rw == m[None]
        amax = jnp.min(jnp.where(eq, iota, E), axis=0)
        hit = iota == amax[None]
        sel = jnp.sum(jnp.where(hit, acc, 0.0), axis=0)
        sel_ref[0, k] = sel.astype(jnp.float32) * m
        map_ref[0, k] = m
        idx_ref[0, k] = amax
        if k + 1 < K:
            rw = jnp.where(hit, neg_inf, rw)


def kernel(x, routing_weights, Wc):
    B, C, H, W = x.shape
    E = Wc.shape[0]
    K = 8
    HW = H * W
    h_t = 32 if H % 32 == 0 else H
    jblocks = H // h_t
    seg = h_t * W
    pad = ((W + 2 + 127) // 128) * 128
    slab = seg + 2 * pad
    nsteps = B * jblocks

    x_flat = x.reshape(B, C, HW)
    rw_flat = routing_weights.reshape(B, E, HW)
    wall = jnp.transpose(Wc, (2, 3, 0, 1)).reshape(9 * E, C)
    wall = wall.astype(jnp.bfloat16)

    body = functools.partial(
        _smoe_kernel, jblocks=jblocks, nsteps=nsteps,
        seg=seg, slab=slab, pad=pad, W=W, E=E, K=K, C=C)

    out_shape = [
        jax.ShapeDtypeStruct((B, K, HW), jnp.float32),
        jax.ShapeDtypeStruct((B, K, HW), jnp.float32),
        jax.ShapeDtypeStruct((B, K, HW), jnp.int32),
    ]
    out_spec = pl.BlockSpec(
        (1, K, seg), lambda s: (s // jblocks, 0, s % jblocks))

    sel, rmap, ridx = pl.pallas_call(
        body,
        grid=(nsteps,),
        in_specs=[
            pl.BlockSpec(memory_space=pl.ANY),
            pl.BlockSpec((1, E, seg),
                         lambda s: (s // jblocks, 0, s % jblocks)),
            pl.BlockSpec((9 * E, C), lambda s: (0, 0)),
        ],
        out_specs=[out_spec, out_spec, out_spec],
        out_shape=out_shape,
        scratch_shapes=[
            pltpu.VMEM((2, C, slab), jnp.float32),
            pltpu.VMEM((9 * E, slab), jnp.bfloat16),
            pltpu.SemaphoreType.DMA((2, 4)),
        ],
        compiler_params=pltpu.CompilerParams(
            dimension_semantics=("arbitrary",)),
    )(x_flat, rw_flat, wall)
    shape4 = (B, K, H, W)
    return (sel.reshape(shape4), rmap.reshape(shape4), ridx.reshape(shape4))


# speedup vs baseline: 1.0803x; 1.0803x over previous
import functools

import jax
import jax.numpy as jnp
from jax import lax
from jax.experimental import pallas as pl
from jax.experimental.pallas import tpu as pltpu


def _smoe_kernel(x_hbm, rw_ref, wall_ref, sel_ref, map_ref, idx_ref,
                 xbuf, pp_ref, sems, *,
                 jblocks, nsteps, seg, slab, pad, W, E, K, C):
    s = pl.program_id(0)
    edge = slab - pad

    nchunk = sems.shape[1]
    cc = C // nchunk

    def dma(t, go):
        b = t // jblocks
        j = t - b * jblocks
        slot = t % 2
        first = j == 0
        last = (j == jblocks - 1) & jnp.logical_not(first)

        @pl.when(first)
        def _():
            for i in range(nchunk):
                cp = pltpu.make_async_copy(
                    x_hbm.at[b, pl.ds(i * cc, cc), pl.ds(0, edge)],
                    xbuf.at[slot, pl.ds(i * cc, cc), pl.ds(pad, edge)],
                    sems.at[slot, i])
                cp.start() if go else cp.wait()

        @pl.when(last)
        def _():
            for i in range(nchunk):
                cp = pltpu.make_async_copy(
                    x_hbm.at[b, pl.ds(i * cc, cc), pl.ds(pl.multiple_of(jnp.maximum(j * seg - pad, 0), 128), edge)],
                    xbuf.at[slot, pl.ds(i * cc, cc), pl.ds(0, edge)],
                    sems.at[slot, i])
                cp.start() if go else cp.wait()

        @pl.when(jnp.logical_not(first | last))
        def _():
            for i in range(nchunk):
                cp = pltpu.make_async_copy(
                    x_hbm.at[b, pl.ds(i * cc, cc), pl.ds(pl.multiple_of(jnp.maximum(j * seg - pad, 0), 128), slab)],
                    xbuf.at[slot, pl.ds(i * cc, cc)],
                    sems.at[slot, i])
                cp.start() if go else cp.wait()

    @pl.when(s == 0)
    def _prologue():
        dma(0, True)

    @pl.when(s + 1 < nsteps)
    def _prefetch():
        dma(s + 1, True)

    dma(s, False)

    xs = xbuf[s % 2].astype(jnp.bfloat16)
    pp_ref[...] = jnp.dot(wall_ref[...], xs,
                          preferred_element_type=jnp.float32)

    j = s % jblocks
    pio = lax.broadcasted_iota(jnp.int32, (1, seg), 1)
    wpos = pio % W
    top_bad = (pio < W) & (j == 0)
    bot_bad = (pio >= seg - W) & (j == jblocks - 1)
    acc = None
    for t in range(9):
        dy, dx = divmod(t, 3)
        off = dy * W + dx + pad - W - 1
        part = pp_ref[t * E:(t + 1) * E, off:off + seg]
        if dx == 0:
            part = jnp.where(wpos == 0, 0.0, part)
        elif dx == 2:
            part = jnp.where(wpos == W - 1, 0.0, part)
        if dy == 0:
            part = jnp.where(top_bad, 0.0, part)
        elif dy == 2:
            part = jnp.where(bot_bad, 0.0, part)
        acc = part if acc is None else acc + part
    accb = acc.astype(jnp.bfloat16)

    rw = rw_ref[0]
    iota = lax.broadcasted_iota(jnp.int32, (E, seg), 0)
    neg_inf = jnp.float32(-jnp.inf)
    for k in range(K):
        m = jnp.max(rw, axis=0)
        eq = rw == m[None]
        amax = jnp.min(jnp.where(eq, iota, E), axis=0)
        hit = iota == amax[None]
        sel = jnp.sum(jnp.where(hit, accb, 0.0), axis=0)
        sel_ref[0, k] = sel.astype(jnp.float32) * m
        map_ref[0, k] = m
        idx_ref[0, k] = amax
        if k + 1 < K:
            rw = jnp.where(hit, neg_inf, rw)


def kernel(x, routing_weights, Wc):
    B, C, H, W = x.shape
    E = Wc.shape[0]
    K = 8
    HW = H * W
    h_t = 32 if H % 32 == 0 else H
    jblocks = H // h_t
    seg = h_t * W
    pad = ((W + 2 + 127) // 128) * 128
    slab = seg + 2 * pad
    nsteps = B * jblocks

    x_flat = x.reshape(B, C, HW)
    rw_flat = routing_weights.reshape(B, E, HW)
    wall = jnp.transpose(Wc, (2, 3, 0, 1)).reshape(9 * E, C)
    wall = wall.astype(jnp.bfloat16)

    body = functools.partial(
        _smoe_kernel, jblocks=jblocks, nsteps=nsteps,
        seg=seg, slab=slab, pad=pad, W=W, E=E, K=K, C=C)

    out_shape = [
        jax.ShapeDtypeStruct((B, K, HW), jnp.float32),
        jax.ShapeDtypeStruct((B, K, HW), jnp.float32),
        jax.ShapeDtypeStruct((B, K, HW), jnp.int32),
    ]
    out_spec = pl.BlockSpec(
        (1, K, seg), lambda s: (s // jblocks, 0, s % jblocks))

    sel, rmap, ridx = pl.pallas_call(
        body,
        grid=(nsteps,),
        in_specs=[
            pl.BlockSpec(memory_space=pl.ANY),
            pl.BlockSpec((1, E, seg),
                         lambda s: (s // jblocks, 0, s % jblocks)),
            pl.BlockSpec((9 * E, C), lambda s: (0, 0)),
        ],
        out_specs=[out_spec, out_spec, out_spec],
        out_shape=out_shape,
        scratch_shapes=[
            pltpu.VMEM((2, C, slab), jnp.float32),
            pltpu.VMEM((9 * E, slab), jnp.float32),
            pltpu.SemaphoreType.DMA((2, 4)),
        ],
        compiler_params=pltpu.CompilerParams(
            dimension_semantics=("arbitrary",)),
    )(x_flat, rw_flat, wall)
    shape4 = (B, K, H, W)
    return (sel.reshape(shape4), rmap.reshape(shape4), ridx.reshape(shape4))


# speedup vs baseline: 1.1074x; 1.0251x over previous
import functools

import jax
import jax.numpy as jnp
from jax import lax
from jax.experimental import pallas as pl
from jax.experimental.pallas import tpu as pltpu


def _smoe_kernel(x_hbm, rw_ref, wall_ref, sel_ref, map_ref, idx_ref,
                 xbuf, pp_ref, sems, *,
                 jblocks, nsteps, seg, slab, pad, W, E, K, C):
    s = pl.program_id(0)
    edge = slab - pad

    nchunk = sems.shape[1]
    cc = C // nchunk

    def dma(t, go):
        b = t // jblocks
        j = t - b * jblocks
        slot = t % 2
        first = j == 0
        last = (j == jblocks - 1) & jnp.logical_not(first)

        @pl.when(first)
        def _():
            for i in range(nchunk):
                cp = pltpu.make_async_copy(
                    x_hbm.at[b, pl.ds(i * cc, cc), pl.ds(0, edge)],
                    xbuf.at[slot, pl.ds(i * cc, cc), pl.ds(pad, edge)],
                    sems.at[slot, i])
                cp.start() if go else cp.wait()

        @pl.when(last)
        def _():
            for i in range(nchunk):
                cp = pltpu.make_async_copy(
                    x_hbm.at[b, pl.ds(i * cc, cc), pl.ds(pl.multiple_of(jnp.maximum(j * seg - pad, 0), 128), edge)],
                    xbuf.at[slot, pl.ds(i * cc, cc), pl.ds(0, edge)],
                    sems.at[slot, i])
                cp.start() if go else cp.wait()

        @pl.when(jnp.logical_not(first | last))
        def _():
            for i in range(nchunk):
                cp = pltpu.make_async_copy(
                    x_hbm.at[b, pl.ds(i * cc, cc), pl.ds(pl.multiple_of(jnp.maximum(j * seg - pad, 0), 128), slab)],
                    xbuf.at[slot, pl.ds(i * cc, cc)],
                    sems.at[slot, i])
                cp.start() if go else cp.wait()

    @pl.when(s == 0)
    def _prologue():
        dma(0, True)

    @pl.when(s + 1 < nsteps)
    def _prefetch():
        dma(s + 1, True)

    dma(s, False)

    xs = xbuf[s % 2].astype(jnp.bfloat16)
    pp = jnp.dot(wall_ref[...], xs, preferred_element_type=jnp.float32)

    j = s % jblocks
    pio = lax.broadcasted_iota(jnp.int32, (1, seg), 1)
    wpos = pio % W
    top_bad = (pio < W) & (j == 0)
    bot_bad = (pio >= seg - W) & (j == jblocks - 1)
    acc = None
    for t in range(9):
        dy, dx = divmod(t, 3)
        off = dy * W + dx + pad - W - 1
        part = lax.slice(pp, (t * E, off), ((t + 1) * E, off + seg))
        if dx == 0:
            part = jnp.where(wpos == 0, 0.0, part)
        elif dx == 2:
            part = jnp.where(wpos == W - 1, 0.0, part)
        if dy == 0:
            part = jnp.where(top_bad, 0.0, part)
        elif dy == 2:
            part = jnp.where(bot_bad, 0.0, part)
        acc = part if acc is None else acc + part

    rw = rw_ref[0]
    iota = lax.broadcasted_iota(jnp.int32, (E, seg), 0)
    neg_inf = jnp.float32(-jnp.inf)
    for k in range(K):
        m = jnp.max(rw, axis=0)
        eq = rw == m[None]
        amax = jnp.min(jnp.where(eq, iota, E), axis=0)
        hit = iota == amax[None]
        sel = jnp.sum(jnp.where(hit, acc, 0.0), axis=0)
        sel_ref[0, k] = sel * m
        map_ref[0, k] = m
        idx_ref[0, k] = amax
        if k + 1 < K:
            rw = jnp.where(hit, neg_inf, rw)


def kernel(x, routing_weights, Wc):
    B, C, H, W = x.shape
    E = Wc.shape[0]
    K = 8
    HW = H * W
    h_t = 32 if H % 32 == 0 else H
    jblocks = H // h_t
    seg = h_t * W
    pad = ((W + 2 + 127) // 128) * 128
    slab = seg + 2 * pad
    nsteps = B * jblocks

    x_flat = x.reshape(B, C, HW)
    rw_flat = routing_weights.reshape(B, E, HW)
    wall = jnp.transpose(Wc, (2, 3, 0, 1)).reshape(9 * E, C)
    wall = wall.astype(jnp.bfloat16)

    body = functools.partial(
        _smoe_kernel, jblocks=jblocks, nsteps=nsteps,
        seg=seg, slab=slab, pad=pad, W=W, E=E, K=K, C=C)

    out_shape = [
        jax.ShapeDtypeStruct((B, K, HW), jnp.float32),
        jax.ShapeDtypeStruct((B, K, HW), jnp.float32),
        jax.ShapeDtypeStruct((B, K, HW), jnp.int32),
    ]
    out_spec = pl.BlockSpec(
        (1, K, seg), lambda s: (s // jblocks, 0, s % jblocks))

    sel, rmap, ridx = pl.pallas_call(
        body,
        grid=(nsteps,),
        in_specs=[
            pl.BlockSpec(memory_space=pl.ANY),
            pl.BlockSpec((1, E, seg),
                         lambda s: (s // jblocks, 0, s % jblocks)),
            pl.BlockSpec((9 * E, C), lambda s: (0, 0)),
        ],
        out_specs=[out_spec, out_spec, out_spec],
        out_shape=out_shape,
        scratch_shapes=[
            pltpu.VMEM((2, C, slab), jnp.float32),
            pltpu.VMEM((9 * E, slab), jnp.float32),
            pltpu.SemaphoreType.DMA((2, 4)),
        ],
        compiler_params=pltpu.CompilerParams(
            dimension_semantics=("arbitrary",)),
    )(x_flat, rw_flat, wall)
    shape4 = (B, K, H, W)
    return (sel.reshape(shape4), rmap.reshape(shape4), ridx.reshape(shape4))


# speedup vs baseline: 1.1152x; 1.0071x over previous
import functools

import jax
import jax.numpy as jnp
from jax import lax
from jax.experimental import pallas as pl
from jax.experimental.pallas import tpu as pltpu


def _smoe_kernel(x_hbm, rw_ref, wall_ref, sel_ref, map_ref, idx_ref,
                 xbuf, pp_ref, sems, *,
                 jblocks, nsteps, seg, slab, pad, W, E, K, C):
    s = pl.program_id(0)
    edge = slab - pad

    nchunk = sems.shape[1]
    cc = C // nchunk

    def dma(t, go):
        b = t // jblocks
        j = t - b * jblocks
        slot = t % 2
        first = j == 0
        last = (j == jblocks - 1) & jnp.logical_not(first)

        @pl.when(first)
        def _():
            for i in range(nchunk):
                cp = pltpu.make_async_copy(
                    x_hbm.at[b, pl.ds(i * cc, cc), pl.ds(0, edge)],
                    xbuf.at[slot, pl.ds(i * cc, cc), pl.ds(pad, edge)],
                    sems.at[slot, i])
                cp.start() if go else cp.wait()

        @pl.when(last)
        def _():
            for i in range(nchunk):
                cp = pltpu.make_async_copy(
                    x_hbm.at[b, pl.ds(i * cc, cc), pl.ds(pl.multiple_of(jnp.maximum(j * seg - pad, 0), 128), edge)],
                    xbuf.at[slot, pl.ds(i * cc, cc), pl.ds(0, edge)],
                    sems.at[slot, i])
                cp.start() if go else cp.wait()

        @pl.when(jnp.logical_not(first | last))
        def _():
            for i in range(nchunk):
                cp = pltpu.make_async_copy(
                    x_hbm.at[b, pl.ds(i * cc, cc), pl.ds(pl.multiple_of(jnp.maximum(j * seg - pad, 0), 128), slab)],
                    xbuf.at[slot, pl.ds(i * cc, cc)],
                    sems.at[slot, i])
                cp.start() if go else cp.wait()

    @pl.when(s == 0)
    def _prologue():
        dma(0, True)

    @pl.when(s + 1 < nsteps)
    def _prefetch():
        dma(s + 1, True)

    dma(s, False)

    xs = xbuf[s % 2].astype(jnp.bfloat16)
    pp_ref[...] = jnp.dot(wall_ref[...], xs,
                          preferred_element_type=jnp.float32)

    j = s % jblocks
    pio = lax.broadcasted_iota(jnp.int32, (1, seg), 1)
    wpos = pio % W
    top_bad = (pio < W) & (j == 0)
    bot_bad = (pio >= seg - W) & (j == jblocks - 1)
    acc = None
    for t in range(9):
        dy, dx = divmod(t, 3)
        off = dy * W + dx + pad - W - 1
        part = pp_ref[t * E:(t + 1) * E, off:off + seg]
        if dx == 0:
            part = jnp.where(wpos == 0, 0.0, part)
        elif dx == 2:
            part = jnp.where(wpos == W - 1, 0.0, part)
        if dy == 0:
            part = jnp.where(top_bad, 0.0, part)
        elif dy == 2:
            part = jnp.where(bot_bad, 0.0, part)
        acc = part if acc is None else acc + part

    rw = rw_ref[0]
    iota = lax.broadcasted_iota(jnp.int32, (E, seg), 0)
    neg_inf = jnp.float32(-jnp.inf)
    for k in range(K):
        m = jnp.max(rw, axis=0)
        eq = rw == m[None]
        amax = jnp.min(jnp.where(eq, iota, E), axis=0)
        hit = iota == amax[None]
        sel = jnp.sum(jnp.where(hit, acc, 0.0), axis=0)
        sel_ref[0, k] = sel * m
        map_ref[0, k] = m
        idx_ref[0, k] = amax
        if k + 1 < K:
            rw = jnp.where(hit, neg_inf, rw)


def kernel(x, routing_weights, Wc):
    B, C, H, W = x.shape
    E = Wc.shape[0]
    K = 8
    HW = H * W
    h_t = 56 if H % 56 == 0 else H
    jblocks = H // h_t
    seg = h_t * W
    pad = ((W + 2 + 127) // 128) * 128
    slab = seg + 2 * pad
    nsteps = B * jblocks

    x_flat = x.reshape(B, C, HW)
    rw_flat = routing_weights.reshape(B, E, HW)
    wall = jnp.transpose(Wc, (2, 3, 0, 1)).reshape(9 * E, C)
    wall = wall.astype(jnp.bfloat16)

    body = functools.partial(
        _smoe_kernel, jblocks=jblocks, nsteps=nsteps,
        seg=seg, slab=slab, pad=pad, W=W, E=E, K=K, C=C)

    out_shape = [
        jax.ShapeDtypeStruct((B, K, HW), jnp.float32),
        jax.ShapeDtypeStruct((B, K, HW), jnp.float32),
        jax.ShapeDtypeStruct((B, K, HW), jnp.int32),
    ]
    out_spec = pl.BlockSpec(
        (1, K, seg), lambda s: (s // jblocks, 0, s % jblocks))

    sel, rmap, ridx = pl.pallas_call(
        body,
        grid=(nsteps,),
        in_specs=[
            pl.BlockSpec(memory_space=pl.ANY),
            pl.BlockSpec((1, E, seg),
                         lambda s: (s // jblocks, 0, s % jblocks)),
            pl.BlockSpec((9 * E, C), lambda s: (0, 0)),
        ],
        out_specs=[out_spec, out_spec, out_spec],
        out_shape=out_shape,
        scratch_shapes=[
            pltpu.VMEM((2, C, slab), jnp.float32),
            pltpu.VMEM((9 * E, slab), jnp.float32),
            pltpu.SemaphoreType.DMA((2, 4)),
        ],
        compiler_params=pltpu.CompilerParams(
            dimension_semantics=("arbitrary",)),
    )(x_flat, rw_flat, wall)
    shape4 = (B, K, H, W)
    return (sel.reshape(shape4), rmap.reshape(shape4), ridx.reshape(shape4))
